# Initial kernel scaffold; baseline (speedup 1.0000x reference)
import jax
import jax.numpy as jnp
from jax.experimental import pallas as pl


def kernel(logits, labels):
    raise NotImplementedError("write your pallas kernel here")


# speedup vs baseline: 64.2227x; 64.2227x over previous
import jax
import jax.numpy as jnp
from jax import lax
from jax.experimental import pallas as pl
from jax.experimental.pallas import tpu as pltpu
from jax.experimental.pallas import tpu_sc as plsc

C = 19
NB = 2048
HISTN = 2 * C * NB
NW = 32
CHUNK = 2048


def _bin_kernel(logits_ref, labels_ref, out_ref):
  x = logits_ref[0]
  m = jnp.max(x, axis=0, keepdims=True)
  e = jnp.exp(x - m)
  s = jnp.sum(e, axis=0, keepdims=True)
  p = e / s
  lbl = labels_ref[0]
  cls = lax.broadcasted_iota(jnp.int32, (C, 8, 512), 0)
  fg = lbl[None, :, :] == cls
  err = jnp.where(fg, 1.0 - p, p)
  b = jnp.minimum((err * NB).astype(jnp.int32), NB - 1)
  out_ref[0] = jnp.where(fg, C * NB, 0) + cls * NB + b


def _bin_indices(logits, labels):
  return pl.pallas_call(
      _bin_kernel,
      grid=(4, 64),
      in_specs=[
          pl.BlockSpec((1, C, 8, 512), lambda i, j: (i, 0, j, 0)),
          pl.BlockSpec((1, 8, 512), lambda i, j: (i, j, 0)),
      ],
      out_specs=pl.BlockSpec((1, C, 8, 512), lambda i, j: (i, 0, j, 0)),
      out_shape=jax.ShapeDtypeStruct((4, C, 512, 512), jnp.int32),
  )(logits, labels)


def _hist_body(idx_hbm, out_hbm, buf0, buf1, hist, sem0, sem1):
  nc = 2
  wid = lax.axis_index("s") * nc + lax.axis_index("c")
  per_tile = idx_hbm.shape[0] // NW
  nchunk = per_tile // CHUNK
  base = wid * per_tile

  def zero_body(i, carry):
    hist[pl.ds(pl.multiple_of(i * 16, 16), 16)] = jnp.zeros((16,), jnp.int32)
    return carry
  lax.fori_loop(0, HISTN // 16, zero_body, 0)

  def process(buf):
    def body(j, carry):
      v = buf[pl.ds(pl.multiple_of(j * 16, 16), 16)]
      plsc.addupdate_scatter(hist, [v], jnp.ones((16,), jnp.int32))
      return carry
    lax.fori_loop(0, CHUNK // 16, body, 0)

  pltpu.async_copy(idx_hbm.at[pl.ds(base, CHUNK)], buf0, sem0)

  def outer(t, carry):
    k0 = 2 * t
    pltpu.async_copy(idx_hbm.at[pl.ds(base + (k0 + 1) * CHUNK, CHUNK)],
                     buf1, sem1)
    pltpu.make_async_copy(idx_hbm.at[pl.ds(base, CHUNK)], buf0, sem0).wait()
    process(buf0)

    @pl.when(k0 + 2 < nchunk)
    def _start_next():
      pltpu.async_copy(idx_hbm.at[pl.ds(base + (k0 + 2) * CHUNK, CHUNK)],
                       buf0, sem0)

    pltpu.make_async_copy(idx_hbm.at[pl.ds(base, CHUNK)], buf1, sem1).wait()
    process(buf1)
    return carry
  lax.fori_loop(0, nchunk // 2, outer, 0)

  pltpu.sync_copy(hist, out_hbm.at[wid])


def _histogram(idx_flat):
  mesh = plsc.VectorSubcoreMesh(core_axis_name="c", subcore_axis_name="s")
  return pl.kernel(
      _hist_body,
      out_type=jax.ShapeDtypeStruct((NW, HISTN), jnp.int32),
      mesh=mesh,
      compiler_params=pltpu.CompilerParams(needs_layout_passes=False),
      scratch_types=[
          pltpu.VMEM((CHUNK,), jnp.int32),
          pltpu.VMEM((CHUNK,), jnp.int32),
          pltpu.VMEM((HISTN,), jnp.int32),
          pltpu.SemaphoreType.DMA,
          pltpu.SemaphoreType.DMA,
      ],
  )(idx_flat)


def _final_kernel(parts_ref, out_ref):
  hs = jnp.sum(parts_ref[...], axis=0).astype(jnp.float32)
  x = hs.reshape(2 * C, NB)
  s = 1
  while s < NB:
    x = x + jnp.concatenate(
        [x[:, s:], jnp.zeros((2 * C, s), jnp.float32)], axis=1)
    s *= 2
  ncum = x[:C] + x[C:]
  fcum = x[C:]
  g = fcum[:, 0:1]
  jac = 1.0 - (g - fcum) / jnp.maximum(g + ncum - fcum, 1.0)
  w = jnp.float32(1.0 / NB)
  loss_c = jnp.sum(jac, axis=1) * w - w * 0.5
  present = g[:, 0] > 0
  loss = (jnp.sum(jnp.where(present, loss_c, 0.0))
          / jnp.sum(present.astype(jnp.float32)))
  out_ref[...] = jnp.broadcast_to(loss, (1, 1))


def _finalize(parts):
  return pl.pallas_call(
      _final_kernel,
      out_shape=jax.ShapeDtypeStruct((1, 1), jnp.float32),
  )(parts)


@jax.jit
def kernel(logits, labels):
  idx = _bin_indices(logits, labels)
  parts = _histogram(idx.reshape(-1))
  return _finalize(parts)[0, 0]


# speedup vs baseline: 69.0303x; 1.0749x over previous
import jax
import jax.numpy as jnp
from jax import lax
from jax.experimental import pallas as pl
from jax.experimental.pallas import tpu as pltpu
from jax.experimental.pallas import tpu_sc as plsc

C = 19
NB = 2048
HISTN = 2 * C * NB
NW = 32
CHUNK = 2048


def _bin_kernel(logits_ref, labels_ref, out_ref):
  x = logits_ref[0]
  m = jnp.max(x, axis=0, keepdims=True)
  e = jnp.exp(x - m)
  s = jnp.sum(e, axis=0, keepdims=True)
  p = e / s
  lbl = labels_ref[0]
  cls = lax.broadcasted_iota(jnp.int32, (C, 8, 512), 0)
  fg = lbl[None, :, :] == cls
  err = jnp.where(fg, 1.0 - p, p)
  b = jnp.minimum((err * NB).astype(jnp.int32), NB - 1)
  out_ref[0] = jnp.where(fg, C * NB, 0) + cls * NB + b


def _bin_indices(logits, labels):
  return pl.pallas_call(
      _bin_kernel,
      grid=(4, 64),
      in_specs=[
          pl.BlockSpec((1, C, 8, 512), lambda i, j: (i, 0, j, 0)),
          pl.BlockSpec((1, 8, 512), lambda i, j: (i, j, 0)),
      ],
      out_specs=pl.BlockSpec((1, C, 8, 512), lambda i, j: (i, 0, j, 0)),
      out_shape=jax.ShapeDtypeStruct((4, C, 512, 512), jnp.int32),
  )(logits, labels)


def _hist_body(idx_hbm, out_hbm, buf0, buf1, hist, sem0, sem1):
  nc = 2
  wid = lax.axis_index("s") * nc + lax.axis_index("c")
  per_tile = idx_hbm.shape[0] // NW
  nchunk = per_tile // CHUNK
  base = wid * per_tile

  zeros16 = jnp.zeros((16,), jnp.int32)

  def zero_body(i, carry):
    b = pl.multiple_of(i * 256, 256)
    for u in range(16):
      hist[pl.ds(b + u * 16, 16)] = zeros16
    return carry
  lax.fori_loop(0, HISTN // 256, zero_body, 0)

  ones16 = jnp.ones((16,), jnp.int32)

  def process(buf):
    def body(j, carry):
      b = pl.multiple_of(j * 256, 256)
      for u in range(16):
        v = buf[pl.ds(b + u * 16, 16)]
        plsc.addupdate_scatter(hist, [v], ones16)
      return carry
    lax.fori_loop(0, CHUNK // 256, body, 0)

  pltpu.async_copy(idx_hbm.at[pl.ds(base, CHUNK)], buf0, sem0)

  def outer(t, carry):
    k0 = 2 * t
    pltpu.async_copy(idx_hbm.at[pl.ds(base + (k0 + 1) * CHUNK, CHUNK)],
                     buf1, sem1)
    pltpu.make_async_copy(idx_hbm.at[pl.ds(base, CHUNK)], buf0, sem0).wait()
    process(buf0)

    @pl.when(k0 + 2 < nchunk)
    def _start_next():
      pltpu.async_copy(idx_hbm.at[pl.ds(base + (k0 + 2) * CHUNK, CHUNK)],
                       buf0, sem0)

    pltpu.make_async_copy(idx_hbm.at[pl.ds(base, CHUNK)], buf1, sem1).wait()
    process(buf1)
    return carry
  lax.fori_loop(0, nchunk // 2, outer, 0)

  pltpu.sync_copy(hist, out_hbm.at[wid])


def _histogram(idx_flat):
  mesh = plsc.VectorSubcoreMesh(core_axis_name="c", subcore_axis_name="s")
  return pl.kernel(
      _hist_body,
      out_type=jax.ShapeDtypeStruct((NW, HISTN), jnp.int32),
      mesh=mesh,
      compiler_params=pltpu.CompilerParams(needs_layout_passes=False),
      scratch_types=[
          pltpu.VMEM((CHUNK,), jnp.int32),
          pltpu.VMEM((CHUNK,), jnp.int32),
          pltpu.VMEM((HISTN,), jnp.int32),
          pltpu.SemaphoreType.DMA,
          pltpu.SemaphoreType.DMA,
      ],
  )(idx_flat)


def _final_kernel(parts_ref, out_ref):
  hs = jnp.sum(parts_ref[...], axis=0).astype(jnp.float32)
  x = hs.reshape(2 * C, NB)
  s = 1
  while s < NB:
    x = x + jnp.concatenate(
        [x[:, s:], jnp.zeros((2 * C, s), jnp.float32)], axis=1)
    s *= 2
  ncum = x[:C] + x[C:]
  fcum = x[C:]
  g = fcum[:, 0:1]
  jac = 1.0 - (g - fcum) / jnp.maximum(g + ncum - fcum, 1.0)
  w = jnp.float32(1.0 / NB)
  loss_c = jnp.sum(jac, axis=1) * w - w * 0.5
  present = g[:, 0] > 0
  loss = (jnp.sum(jnp.where(present, loss_c, 0.0))
          / jnp.sum(present.astype(jnp.float32)))
  out_ref[...] = jnp.broadcast_to(loss, (1, 1))


def _finalize(parts):
  return pl.pallas_call(
      _final_kernel,
      out_shape=jax.ShapeDtypeStruct((1, 1), jnp.float32),
  )(parts)


@jax.jit
def kernel(logits, labels):
  idx = _bin_indices(logits, labels)
  parts = _histogram(idx.reshape(-1))
  return _finalize(parts)[0, 0]


# speedup vs baseline: 78.9650x; 1.1439x over previous
import jax
import jax.numpy as jnp
from jax import lax
from jax.experimental import pallas as pl
from jax.experimental.pallas import tpu as pltpu
from jax.experimental.pallas import tpu_sc as plsc

C = 19
NB = 1024
HISTN = 2 * C * NB
NW = 32
ROWS = 4
CHUNK = ROWS * 512
NCHUNK = 76 * 512 // ROWS // NW


def _bin_kernel(logits_ref, labels_ref, out_ref):
  x = logits_ref[0]
  m = jnp.max(x, axis=0, keepdims=True)
  e = jnp.exp(x - m)
  s = jnp.sum(e, axis=0, keepdims=True)
  p = e / s
  lbl = labels_ref[0]
  cls = lax.broadcasted_iota(jnp.int32, (C, 8, 512), 0)
  fg = lbl[None, :, :] == cls
  err = jnp.where(fg, 1.0 - p, p)
  b = jnp.minimum((err * NB).astype(jnp.int32), NB - 1)
  out_ref[...] = jnp.where(fg, C * NB, 0) + cls * NB + b


def _bin_indices(logits, labels):
  return pl.pallas_call(
      _bin_kernel,
      grid=(4, 64),
      in_specs=[
          pl.BlockSpec((1, C, 8, 512), lambda i, j: (i, 0, j, 0)),
          pl.BlockSpec((1, 8, 512), lambda i, j: (i, j, 0)),
      ],
      out_specs=pl.BlockSpec((C, 8, 512), lambda i, j: (i, j, 0)),
      out_shape=jax.ShapeDtypeStruct((4 * C, 512, 512), jnp.int32),
  )(logits, labels)


def _hist_body(idx_hbm, out_hbm, buf0, buf1, h0, h1, sem0, sem1):
  nc = 2
  wid = lax.axis_index("s") * nc + lax.axis_index("c")
  gbase = wid * NCHUNK

  zeros16 = jnp.zeros((16,), jnp.int32)

  def zero_body(i, carry):
    b = pl.multiple_of(i * 256, 256)
    for u in range(16):
      h0[pl.ds(b + u * 16, 16)] = zeros16
      h1[pl.ds(b + u * 16, 16)] = zeros16
    return carry
  lax.fori_loop(0, HISTN // 256, zero_body, 0)

  ones16 = jnp.ones((16,), jnp.int32)

  def start_copy(g, buf, sem):
    p = lax.shift_right_logical(g, 7)
    r = lax.mul(lax.bitwise_and(g, 127), ROWS)
    return pltpu.async_copy(idx_hbm.at[p, pl.ds(r, ROWS), :], buf, sem)

  def process(buf):
    for u in range(ROWS):
      def body(j, carry):
        b = pl.multiple_of(j * 256, 256)
        for k in range(16):
          v = buf[u, pl.ds(b + k * 16, 16)]
          plsc.addupdate_scatter(h0 if k % 2 == 0 else h1, [v], ones16)
        return carry
      lax.fori_loop(0, 512 // 256, body, 0)

  start_copy(gbase, buf0, sem0)

  def outer(t, carry):
    g0 = gbase + 2 * t
    start_copy(g0 + 1, buf1, sem1)
    pltpu.make_async_copy(idx_hbm.at[0, pl.ds(0, ROWS), :], buf0, sem0).wait()
    process(buf0)

    @pl.when(2 * t + 2 < NCHUNK)
    def _start_next():
      start_copy(g0 + 2, buf0, sem0)

    pltpu.make_async_copy(idx_hbm.at[0, pl.ds(0, ROWS), :], buf1, sem1).wait()
    process(buf1)
    return carry
  lax.fori_loop(0, NCHUNK // 2, outer, 0)

  pltpu.sync_copy(h0, out_hbm.at[2 * wid])
  pltpu.sync_copy(h1, out_hbm.at[2 * wid + 1])


def _histogram(idx):
  mesh = plsc.VectorSubcoreMesh(core_axis_name="c", subcore_axis_name="s")
  return pl.kernel(
      _hist_body,
      out_type=jax.ShapeDtypeStruct((2 * NW, HISTN), jnp.int32),
      mesh=mesh,
      compiler_params=pltpu.CompilerParams(needs_layout_passes=False),
      scratch_types=[
          pltpu.VMEM((ROWS, 512), jnp.int32),
          pltpu.VMEM((ROWS, 512), jnp.int32),
          pltpu.VMEM((HISTN,), jnp.int32),
          pltpu.VMEM((HISTN,), jnp.int32),
          pltpu.SemaphoreType.DMA,
          pltpu.SemaphoreType.DMA,
      ],
  )(idx)


def _final_kernel(parts_ref, out_ref):
  hs = jnp.sum(parts_ref[...], axis=0).astype(jnp.float32)
  x = hs.reshape(2 * C, NB)
  s = 1
  while s < NB:
    x = x + jnp.concatenate(
        [x[:, s:], jnp.zeros((2 * C, s), jnp.float32)], axis=1)
    s *= 2
  ncum = x[:C] + x[C:]
  fcum = x[C:]
  g = fcum[:, 0:1]
  jac = 1.0 - (g - fcum) / jnp.maximum(g + ncum - fcum, 1.0)
  w = jnp.float32(1.0 / NB)
  loss_c = jnp.sum(jac, axis=1) * w - w * 0.5
  present = g[:, 0] > 0
  loss = (jnp.sum(jnp.where(present, loss_c, 0.0))
          / jnp.sum(present.astype(jnp.float32)))
  out_ref[...] = jnp.broadcast_to(loss, (1, 1))


def _finalize(parts):
  return pl.pallas_call(
      _final_kernel,
      out_shape=jax.ShapeDtypeStruct((1, 1), jnp.float32),
  )(parts)


@jax.jit
def kernel(logits, labels):
  idx = _bin_indices(logits, labels)
  parts = _histogram(idx)
  return _finalize(parts)[0, 0]


# speedup vs baseline: 102.1982x; 1.2942x over previous
import jax
import jax.numpy as jnp
from jax import lax
from jax.experimental import pallas as pl
from jax.experimental.pallas import tpu as pltpu
from jax.experimental.pallas import tpu_sc as plsc

C = 19
NB = 1024
HISTN = 2 * C * NB
NW = 32
ROWS = 4
CHUNK = ROWS * 512
NCHUNK = 76 * 512 // ROWS // NW


def _bin_kernel(logits_ref, labels_ref, out_ref):
  x = logits_ref[0]
  m = jnp.max(x, axis=0, keepdims=True)
  e = jnp.exp(x - m)
  s = jnp.sum(e, axis=0, keepdims=True)
  p = e / s
  lbl = labels_ref[0]
  cls = lax.broadcasted_iota(jnp.int32, (C, 8, 512), 0)
  fg = lbl[None, :, :] == cls
  err = jnp.where(fg, 1.0 - p, p)
  b = jnp.minimum((err * NB).astype(jnp.int32), NB - 1)
  out_ref[...] = jnp.where(fg, C * NB, 0) + cls * NB + b


def _bin_indices(logits, labels):
  return pl.pallas_call(
      _bin_kernel,
      grid=(4, 64),
      in_specs=[
          pl.BlockSpec((1, C, 8, 512), lambda i, j: (i, 0, j, 0)),
          pl.BlockSpec((1, 8, 512), lambda i, j: (i, j, 0)),
      ],
      out_specs=pl.BlockSpec((C, 8, 512), lambda i, j: (i, j, 0)),
      out_shape=jax.ShapeDtypeStruct((4 * C, 512, 512), jnp.int32),
  )(logits, labels)


def _hist_body(idx_hbm, out_hbm, buf0, buf1, h0, h1, sem0, sem1):
  nc = 2
  wid = lax.axis_index("s") * nc + lax.axis_index("c")
  gbase = wid * NCHUNK

  zeros16 = jnp.zeros((16,), jnp.int32)

  def zero_body(i, carry):
    b = pl.multiple_of(i * 256, 256)
    for u in range(16):
      h0[pl.ds(b + u * 16, 16)] = zeros16
      h1[pl.ds(b + u * 16, 16)] = zeros16
    return carry
  lax.fori_loop(0, HISTN // 256, zero_body, 0)

  ones16 = jnp.ones((16,), jnp.int32)

  def start_copy(g, buf, sem):
    p = lax.shift_right_logical(g, 7)
    r = lax.mul(lax.bitwise_and(g, 127), ROWS)
    return pltpu.async_copy(idx_hbm.at[p, pl.ds(r, ROWS), :], buf, sem)

  def process(buf):
    for u in range(ROWS):
      vs = [buf[u, pl.ds(k * 16, 16)] for k in range(32)]
      for k in range(32):
        plsc.addupdate_scatter(h0 if k % 2 == 0 else h1, [vs[k]], ones16)

  start_copy(gbase, buf0, sem0)

  def outer(t, carry):
    g0 = gbase + 2 * t
    start_copy(g0 + 1, buf1, sem1)
    pltpu.make_async_copy(idx_hbm.at[0, pl.ds(0, ROWS), :], buf0, sem0).wait()
    process(buf0)

    @pl.when(2 * t + 2 < NCHUNK)
    def _start_next():
      start_copy(g0 + 2, buf0, sem0)

    pltpu.make_async_copy(idx_hbm.at[0, pl.ds(0, ROWS), :], buf1, sem1).wait()
    process(buf1)
    return carry
  lax.fori_loop(0, NCHUNK // 2, outer, 0)

  pltpu.sync_copy(h0, out_hbm.at[2 * wid])
  pltpu.sync_copy(h1, out_hbm.at[2 * wid + 1])


def _histogram(idx):
  mesh = plsc.VectorSubcoreMesh(core_axis_name="c", subcore_axis_name="s")
  return pl.kernel(
      _hist_body,
      out_type=jax.ShapeDtypeStruct((2 * NW, HISTN), jnp.int32),
      mesh=mesh,
      compiler_params=pltpu.CompilerParams(needs_layout_passes=False),
      scratch_types=[
          pltpu.VMEM((ROWS, 512), jnp.int32),
          pltpu.VMEM((ROWS, 512), jnp.int32),
          pltpu.VMEM((HISTN,), jnp.int32),
          pltpu.VMEM((HISTN,), jnp.int32),
          pltpu.SemaphoreType.DMA,
          pltpu.SemaphoreType.DMA,
      ],
  )(idx)


def _final_kernel(parts_ref, out_ref):
  hs = jnp.sum(parts_ref[...], axis=0).astype(jnp.float32)
  x = hs.reshape(2 * C, NB)
  s = 1
  while s < NB:
    x = x + jnp.concatenate(
        [x[:, s:], jnp.zeros((2 * C, s), jnp.float32)], axis=1)
    s *= 2
  ncum = x[:C] + x[C:]
  fcum = x[C:]
  g = fcum[:, 0:1]
  jac = 1.0 - (g - fcum) / jnp.maximum(g + ncum - fcum, 1.0)
  w = jnp.float32(1.0 / NB)
  loss_c = jnp.sum(jac, axis=1) * w - w * 0.5
  present = g[:, 0] > 0
  loss = (jnp.sum(jnp.where(present, loss_c, 0.0))
          / jnp.sum(present.astype(jnp.float32)))
  out_ref[...] = jnp.broadcast_to(loss, (1, 1))


def _finalize(parts):
  return pl.pallas_call(
      _final_kernel,
      out_shape=jax.ShapeDtypeStruct((1, 1), jnp.float32),
  )(parts)


@jax.jit
def kernel(logits, labels):
  idx = _bin_indices(logits, labels)
  parts = _histogram(idx)
  return _finalize(parts)[0, 0]
